# baseline (speedup 1.0000x reference)
import jax
import jax.numpy as jnp
from jax import lax
from jax.experimental import pallas as pl
from jax.experimental.pallas import tpu as pltpu
from jax.experimental.pallas import tpu_sc as plsc

EMBED = 768
GRID = 16
B = 4
L = GRID * GRID * GRID + 1
NROW = 33
NCOL = 128
LPAD = NROW * NCOL
NLANE = 16
BLK = 32
WPB = 16
NBLKW = 8
NOUTER = NBLKW // 2
IDX_SPAN = 640
NCHUNK = EMBED // NLANE


def _index_body(pol_ref, idx_ref):
  bgrid = pl.program_id(0)
  m = (pol_ref[0] > 0).astype(jnp.float32)

  li = lax.broadcasted_iota(jnp.int32, (NCOL, NCOL), 0)
  lj = lax.broadcasted_iota(jnp.int32, (NCOL, NCOL), 1)
  tri = (li <= lj).astype(jnp.float32)
  ri = lax.broadcasted_iota(jnp.int32, (NROW, NROW), 0)
  rj = lax.broadcasted_iota(jnp.int32, (NROW, NROW), 1)
  stri = (rj < ri).astype(jnp.float32)

  within = jnp.dot(m, tri, preferred_element_type=jnp.float32)
  rowsum = within[:, NCOL - 1:NCOL]
  offs = jnp.dot(stri, rowsum, preferred_element_type=jnp.float32)
  inc = within + offs
  z_total = inc[NROW - 1:NROW, NCOL - 1:NCOL]

  jpos = (lax.broadcasted_iota(jnp.int32, (NROW, NCOL), 0) * NCOL
          + lax.broadcasted_iota(jnp.int32, (NROW, NCOL), 1)).astype(jnp.float32)
  exc = inc - m
  hid = z_total + jpos - inc
  srcf = jnp.where(m > 0, exc, hid)
  srcf = jnp.minimum(srcf, float(L - 1))
  del bgrid
  idx_ref[0] = srcf.astype(jnp.int32)


_index = pl.pallas_call(
    _index_body,
    grid=(B,),
    in_specs=[pl.BlockSpec((1, NROW, NCOL), lambda b: (b, 0, 0))],
    out_specs=pl.BlockSpec((1, NROW, NCOL), lambda b: (b, 0, 0)),
    out_shape=jax.ShapeDtypeStruct((B, NROW, NCOL), jnp.int32),
)


def _gather_body(off, x_hbm, idx_hbm, lt_hbm, pos_hbm, out_hbm,
                 idx_v, lt_v, xb0, xb1, pb0, pb1,
                 sg0, sg1, sp0, sp1, so0, so1):
  cid = lax.axis_index("c")
  sid = lax.axis_index("s")
  wid = sid * 2 + cid
  bl = wid // WPB
  b = bl + off
  k = wid % WPB
  j0 = k * NBLKW * BLK

  xbufs = (xb0, xb1)
  pbufs = (pb0, pb1)
  sgs = (sg0, sg1)
  sps = (sp0, sp1)
  sos = (so0, so1)

  pltpu.sync_copy(idx_hbm.at[b], idx_v)
  pltpu.sync_copy(lt_hbm.at[0, 0, :], lt_v)

  def gather_desc(blk, u):
    jj = j0 + blk * BLK
    return pltpu.make_async_copy(
        x_hbm.at[b].at[idx_v.at[jj // NCOL, pl.ds(jj % NCOL, BLK)]],
        xbufs[u], sgs[u])

  def pos_desc(blk, u):
    return pltpu.make_async_copy(
        pos_hbm.at[0, pl.ds(j0 + blk * BLK, BLK), :], pbufs[u], sps[u])

  def out_desc(blk, u):
    return pltpu.make_async_copy(
        xbufs[u], out_hbm.at[bl, pl.ds(j0 + blk * BLK, BLK), :], sos[u])

  def compute(u):
    xbuf, pbuf = xbufs[u], pbufs[u]

    def col(c, _):
      lo = c * NLANE
      ltc = lt_v[pl.ds(lo, NLANE)]
      for r in range(BLK):
        xbuf[r, pl.ds(lo, NLANE)] = (
            xbuf[r, pl.ds(lo, NLANE)] + pbuf[r, pl.ds(lo, NLANE)] + ltc)
      return 0

    lax.fori_loop(0, NCHUNK, col, 0)

  gather_desc(0, 0).start()
  pos_desc(0, 0).start()

  def outer(g, _):
    blk0 = g * 2
    gather_desc(blk0, 0).wait()
    pos_desc(blk0, 0).wait()

    @pl.when(g >= 1)
    def _():
      out_desc(blk0 - 1, 1).wait()
    gather_desc(blk0 + 1, 1).start()
    pos_desc(blk0 + 1, 1).start()
    compute(0)
    out_desc(blk0, 0).start()

    gather_desc(blk0 + 1, 1).wait()
    pos_desc(blk0 + 1, 1).wait()
    out_desc(blk0, 0).wait()

    @pl.when(g < NOUTER - 1)
    def _():
      gather_desc(blk0 + 2, 0).start()
      pos_desc(blk0 + 2, 0).start()
    compute(1)
    out_desc(blk0 + 1, 1).start()
    return 0

  lax.fori_loop(0, NOUTER, outer, 0)
  out_desc(NBLKW - 1, 1).wait()

  @pl.when(k == WPB - 1)
  def _():
    tail = pltpu.make_async_copy(
        x_hbm.at[b].at[idx_v.at[(L - 1) // NCOL, pl.ds(0, NLANE)]],
        xb0.at[pl.ds(0, NLANE), :], sg0)
    tail.start()
    pltpu.sync_copy(pos_hbm.at[0, pl.ds(L - 1, 1), :], pb0.at[pl.ds(0, 1), :])
    tail.wait()

    def col(c, _):
      lo = c * NLANE
      xb0[0, pl.ds(lo, NLANE)] = (
          xb0[0, pl.ds(lo, NLANE)] + pb0[0, pl.ds(lo, NLANE)]
          + lt_v[pl.ds(lo, NLANE)])
      return 0

    lax.fori_loop(0, NCHUNK, col, 0)
    pltpu.sync_copy(xb0.at[pl.ds(0, 1), :],
                    out_hbm.at[bl, pl.ds(L - 1, 1), :])


import functools as _ft

def _make_gather(off):
  return pl.kernel(
    _ft.partial(_gather_body, off),
    out_type=jax.ShapeDtypeStruct((2, L, EMBED), jnp.float32),
    mesh=plsc.VectorSubcoreMesh(core_axis_name="c", subcore_axis_name="s"),
    compiler_params=pltpu.CompilerParams(use_tc_tiling_on_sc=True),
    scratch_types=[
        pltpu.VMEM((NROW, NCOL), jnp.int32),
        pltpu.VMEM((EMBED,), jnp.float32),
        pltpu.VMEM((BLK, EMBED), jnp.float32),
        pltpu.VMEM((BLK, EMBED), jnp.float32),
        pltpu.VMEM((BLK, EMBED), jnp.float32),
        pltpu.VMEM((BLK, EMBED), jnp.float32),
        pltpu.SemaphoreType.DMA,
        pltpu.SemaphoreType.DMA,
        pltpu.SemaphoreType.DMA,
        pltpu.SemaphoreType.DMA,
        pltpu.SemaphoreType.DMA,
        pltpu.SemaphoreType.DMA,
    ],
  )


_gather_a = _make_gather(0)
_gather_b = _make_gather(2)


@jax.jit
def kernel(x_list, policy_list, layer_token0, pos_embed):
  pol = jnp.pad(policy_list, ((0, 0), (0, LPAD - L))).reshape(B, NROW, NCOL)
  idx = _index(pol)
  oa = _gather_a(x_list, idx, layer_token0, pos_embed)
  ob = _gather_b(x_list, idx, layer_token0, pos_embed)
  return jnp.concatenate([oa, ob], axis=0)


# speedup vs baseline: 1.5403x; 1.5403x over previous
import jax
import jax.numpy as jnp
from jax import lax
from jax.experimental import pallas as pl
from jax.experimental.pallas import tpu as pltpu
from jax.experimental.pallas import tpu_sc as plsc

EMBED = 768
GRID = 16
B = 4
L = GRID * GRID * GRID + 1
NROW = 33
NCOL = 128
LPAD = NROW * NCOL
NLANE = 16
BLK = 32
WPB = 8
NBLKW = 16
NOUTER = NBLKW // 2
IDX_SPAN = 640
NCHUNK = EMBED // NLANE


def _index_body(pol_ref, idx_ref):
  bgrid = pl.program_id(0)
  m = (pol_ref[0] > 0).astype(jnp.float32)

  li = lax.broadcasted_iota(jnp.int32, (NCOL, NCOL), 0)
  lj = lax.broadcasted_iota(jnp.int32, (NCOL, NCOL), 1)
  tri = (li <= lj).astype(jnp.float32)
  ri = lax.broadcasted_iota(jnp.int32, (NROW, NROW), 0)
  rj = lax.broadcasted_iota(jnp.int32, (NROW, NROW), 1)
  stri = (rj < ri).astype(jnp.float32)

  within = jnp.dot(m, tri, preferred_element_type=jnp.float32)
  rowsum = within[:, NCOL - 1:NCOL]
  offs = jnp.dot(stri, rowsum, preferred_element_type=jnp.float32)
  inc = within + offs
  z_total = inc[NROW - 1:NROW, NCOL - 1:NCOL]

  jpos = (lax.broadcasted_iota(jnp.int32, (NROW, NCOL), 0) * NCOL
          + lax.broadcasted_iota(jnp.int32, (NROW, NCOL), 1)).astype(jnp.float32)
  exc = inc - m
  hid = z_total + jpos - inc
  srcf = jnp.where(m > 0, exc, hid)
  srcf = jnp.minimum(srcf, float(L - 1))
  del bgrid
  idx_ref[0] = srcf.astype(jnp.int32)


_index = pl.pallas_call(
    _index_body,
    grid=(B,),
    in_specs=[pl.BlockSpec((1, NROW, NCOL), lambda b: (b, 0, 0))],
    out_specs=pl.BlockSpec((1, NROW, NCOL), lambda b: (b, 0, 0)),
    out_shape=jax.ShapeDtypeStruct((B, NROW, NCOL), jnp.int32),
)


def _gather_body(x_hbm, idx_hbm, lt_hbm, pos_hbm, out_hbm,
                 idx_v, lt_v, xb0, xb1, pb0, pb1,
                 sg0, sg1, sp0, sp1, so0, so1):
  cid = lax.axis_index("c")
  sid = lax.axis_index("s")
  wid = sid * 2 + cid
  b = wid // WPB
  k = wid % WPB
  j0 = k * NBLKW * BLK

  xbufs = (xb0, xb1)
  pbufs = (pb0, pb1)
  sgs = (sg0, sg1)
  sps = (sp0, sp1)
  sos = (so0, so1)

  pltpu.sync_copy(idx_hbm.at[b], idx_v)
  pltpu.sync_copy(lt_hbm.at[0, 0, :], lt_v)

  def gather_desc(blk, u):
    jj = j0 + blk * BLK
    return pltpu.make_async_copy(
        x_hbm.at[b].at[idx_v.at[jj // NCOL, pl.ds(jj % NCOL, BLK)]],
        xbufs[u], sgs[u])

  def pos_desc(blk, u):
    return pltpu.make_async_copy(
        pos_hbm.at[0, pl.ds(j0 + blk * BLK, BLK), :], pbufs[u], sps[u])

  def out_desc(blk, u):
    return pltpu.make_async_copy(
        xbufs[u], out_hbm.at[b, pl.ds(j0 + blk * BLK, BLK), :], sos[u])

  def compute(u):
    xbuf, pbuf = xbufs[u], pbufs[u]

    def col(c, _):
      lo = c * NLANE * 2
      lo2 = lo + NLANE
      ltc = lt_v[pl.ds(lo, NLANE)]
      ltc2 = lt_v[pl.ds(lo2, NLANE)]
      for r in range(BLK):
        xbuf[r, pl.ds(lo, NLANE)] = (
            xbuf[r, pl.ds(lo, NLANE)] + pbuf[r, pl.ds(lo, NLANE)] + ltc)
        xbuf[r, pl.ds(lo2, NLANE)] = (
            xbuf[r, pl.ds(lo2, NLANE)] + pbuf[r, pl.ds(lo2, NLANE)] + ltc2)
      return 0

    lax.fori_loop(0, NCHUNK // 2, col, 0)

  gather_desc(0, 0).start()
  pos_desc(0, 0).start()

  def outer(g, _):
    blk0 = g * 2
    gather_desc(blk0, 0).wait()
    pos_desc(blk0, 0).wait()

    @pl.when(g >= 1)
    def _():
      out_desc(blk0 - 1, 1).wait()
    gather_desc(blk0 + 1, 1).start()
    pos_desc(blk0 + 1, 1).start()
    compute(0)
    out_desc(blk0, 0).start()

    gather_desc(blk0 + 1, 1).wait()
    pos_desc(blk0 + 1, 1).wait()
    out_desc(blk0, 0).wait()

    @pl.when(g < NOUTER - 1)
    def _():
      gather_desc(blk0 + 2, 0).start()
      pos_desc(blk0 + 2, 0).start()
    compute(1)
    out_desc(blk0 + 1, 1).start()
    return 0

  lax.fori_loop(0, NOUTER, outer, 0)
  out_desc(NBLKW - 1, 1).wait()

  @pl.when(k == WPB - 1)
  def _():
    tail = pltpu.make_async_copy(
        x_hbm.at[b].at[idx_v.at[(L - 1) // NCOL, pl.ds(0, NLANE)]],
        xb0.at[pl.ds(0, NLANE), :], sg0)
    tail.start()
    pltpu.sync_copy(pos_hbm.at[0, pl.ds(L - 1, 1), :], pb0.at[pl.ds(0, 1), :])
    tail.wait()

    def col(c, _):
      lo = c * NLANE
      xb0[0, pl.ds(lo, NLANE)] = (
          xb0[0, pl.ds(lo, NLANE)] + pb0[0, pl.ds(lo, NLANE)]
          + lt_v[pl.ds(lo, NLANE)])
      return 0

    lax.fori_loop(0, NCHUNK, col, 0)
    pltpu.sync_copy(xb0.at[pl.ds(0, 1), :],
                    out_hbm.at[b, pl.ds(L - 1, 1), :])


_gather = pl.kernel(
    _gather_body,
    out_type=jax.ShapeDtypeStruct((B, L, EMBED), jnp.float32),
    mesh=plsc.VectorSubcoreMesh(core_axis_name="c", subcore_axis_name="s"),
    compiler_params=pltpu.CompilerParams(use_tc_tiling_on_sc=True),
    scratch_types=[
        pltpu.VMEM((NROW, NCOL), jnp.int32),
        pltpu.VMEM((EMBED,), jnp.float32),
        pltpu.VMEM((BLK, EMBED), jnp.float32),
        pltpu.VMEM((BLK, EMBED), jnp.float32),
        pltpu.VMEM((BLK, EMBED), jnp.float32),
        pltpu.VMEM((BLK, EMBED), jnp.float32),
        pltpu.SemaphoreType.DMA,
        pltpu.SemaphoreType.DMA,
        pltpu.SemaphoreType.DMA,
        pltpu.SemaphoreType.DMA,
        pltpu.SemaphoreType.DMA,
        pltpu.SemaphoreType.DMA,
    ],
)


@jax.jit
def kernel(x_list, policy_list, layer_token0, pos_embed):
  pol = jnp.pad(policy_list, ((0, 0), (0, LPAD - L))).reshape(B, NROW, NCOL)
  idx = _index(pol)
  return _gather(x_list, idx, layer_token0, pos_embed)


# speedup vs baseline: 2.4218x; 1.5723x over previous
import jax
import jax.numpy as jnp
from jax import lax
from jax.experimental import pallas as pl
from jax.experimental.pallas import tpu as pltpu
from jax.experimental.pallas import tpu_sc as plsc

EMBED = 768
GRID = 16
B = 4
L = GRID * GRID * GRID + 1
NROW = 33
NCOL = 128
LPAD = NROW * NCOL
NLANE = 16
BLK = 32
WPB = 8
NBLKW = 16
NOUTER = NBLKW // 2
IDX_SPAN = 640
NCHUNK = EMBED // NLANE


def _index_body(pol_ref, idx_ref):
  bgrid = pl.program_id(0)
  m = (pol_ref[0] > 0).astype(jnp.float32)

  li = lax.broadcasted_iota(jnp.int32, (NCOL, NCOL), 0)
  lj = lax.broadcasted_iota(jnp.int32, (NCOL, NCOL), 1)
  tri = (li <= lj).astype(jnp.float32)
  ri = lax.broadcasted_iota(jnp.int32, (NROW, NROW), 0)
  rj = lax.broadcasted_iota(jnp.int32, (NROW, NROW), 1)
  stri = (rj < ri).astype(jnp.float32)

  within = jnp.dot(m, tri, preferred_element_type=jnp.float32)
  rowsum = within[:, NCOL - 1:NCOL]
  offs = jnp.dot(stri, rowsum, preferred_element_type=jnp.float32)
  inc = within + offs
  z_total = inc[NROW - 1:NROW, NCOL - 1:NCOL]

  jpos = (lax.broadcasted_iota(jnp.int32, (NROW, NCOL), 0) * NCOL
          + lax.broadcasted_iota(jnp.int32, (NROW, NCOL), 1)).astype(jnp.float32)
  exc = inc - m
  hid = z_total + jpos - inc
  srcf = jnp.where(m > 0, exc, hid)
  srcf = jnp.minimum(srcf, float(L - 1))
  del bgrid
  idx_ref[0] = srcf.astype(jnp.int32)


_index = pl.pallas_call(
    _index_body,
    grid=(B,),
    in_specs=[pl.BlockSpec((1, NROW, NCOL), lambda b: (b, 0, 0))],
    out_specs=pl.BlockSpec((1, NROW, NCOL), lambda b: (b, 0, 0)),
    out_shape=jax.ShapeDtypeStruct((B, NROW, NCOL), jnp.int32),
)


def _gather_body(x_hbm, idx_hbm, lt_hbm, pos_hbm, out_hbm,
                 idx_v, lt_v, xb0, xb1, pb0, pb1,
                 sg0, sg1, sp0, sp1, so0, so1):
  cid = lax.axis_index("c")
  sid = lax.axis_index("s")
  wid = sid * 2 + cid
  b = wid // WPB
  k = wid % WPB
  j0 = k * NBLKW * BLK

  xbufs = (xb0, xb1)
  pbufs = (pb0, pb1)
  sgs = (sg0, sg1)
  sps = (sp0, sp1)
  sos = (so0, so1)

  pltpu.sync_copy(idx_hbm.at[b], idx_v)
  pltpu.sync_copy(lt_hbm.at[0, 0, :], lt_v)

  def gather_desc(blk, u):
    jj = j0 + blk * BLK
    return pltpu.make_async_copy(
        x_hbm.at[b].at[idx_v.at[jj // NCOL, pl.ds(jj % NCOL, BLK)]],
        xbufs[u], sgs[u])

  def pos_desc(blk, u):
    return pltpu.make_async_copy(
        pos_hbm.at[0, pl.ds(j0 + blk * BLK, BLK), :], pbufs[u], sps[u])

  def out_desc(blk, u):
    return pltpu.make_async_copy(
        xbufs[u], out_hbm.at[b, pl.ds(j0 + blk * BLK, BLK), :], sos[u])

  def compute(u):
    xbuf, pbuf = xbufs[u], pbufs[u]

    def col(c, _):
      lo = c * NLANE
      ltc = lt_v[pl.ds(lo, NLANE)]
      for r in range(BLK):
        xbuf[r, pl.ds(lo, NLANE)] = (
            xbuf[r, pl.ds(lo, NLANE)] + pbuf[r, pl.ds(lo, NLANE)] + ltc)
      return 0

    lax.fori_loop(0, NCHUNK, col, 0)

  gather_desc(0, 0).start()
  pos_desc(0, 0).start()

  def outer(g, _):
    blk0 = g * 2
    gather_desc(blk0, 0).wait()
    pos_desc(blk0, 0).wait()

    @pl.when(g >= 1)
    def _():
      out_desc(blk0 - 1, 1).wait()
    gather_desc(blk0 + 1, 1).start()
    pos_desc(blk0 + 1, 1).start()
    compute(0)
    out_desc(blk0, 0).start()

    gather_desc(blk0 + 1, 1).wait()
    pos_desc(blk0 + 1, 1).wait()
    out_desc(blk0, 0).wait()

    @pl.when(g < NOUTER - 1)
    def _():
      gather_desc(blk0 + 2, 0).start()
      pos_desc(blk0 + 2, 0).start()
    compute(1)
    out_desc(blk0 + 1, 1).start()
    return 0

  lax.fori_loop(0, NOUTER, outer, 0)
  out_desc(NBLKW - 1, 1).wait()

  @pl.when(k == WPB - 1)
  def _():
    tail = pltpu.make_async_copy(
        x_hbm.at[b].at[idx_v.at[(L - 1) // NCOL, pl.ds(0, NLANE)]],
        xb0.at[pl.ds(0, NLANE), :], sg0)
    tail.start()
    pltpu.sync_copy(pos_hbm.at[0, pl.ds(L - 1, 1), :], pb0.at[pl.ds(0, 1), :])
    tail.wait()

    def col(c, _):
      lo = c * NLANE
      xb0[0, pl.ds(lo, NLANE)] = (
          xb0[0, pl.ds(lo, NLANE)] + pb0[0, pl.ds(lo, NLANE)]
          + lt_v[pl.ds(lo, NLANE)])
      return 0

    lax.fori_loop(0, NCHUNK, col, 0)
    pltpu.sync_copy(xb0.at[pl.ds(0, 1), :],
                    out_hbm.at[b, pl.ds(L - 1, 1), :])


_gather = pl.kernel(
    _gather_body,
    out_type=jax.ShapeDtypeStruct((B, L, EMBED), jnp.float32),
    mesh=plsc.VectorSubcoreMesh(core_axis_name="c", subcore_axis_name="s"),
    compiler_params=pltpu.CompilerParams(use_tc_tiling_on_sc=True),
    scratch_types=[
        pltpu.VMEM((NROW, NCOL), jnp.int32),
        pltpu.VMEM((EMBED,), jnp.float32),
        pltpu.VMEM((BLK, EMBED), jnp.float32),
        pltpu.VMEM((BLK, EMBED), jnp.float32),
        pltpu.VMEM((BLK, EMBED), jnp.float32),
        pltpu.VMEM((BLK, EMBED), jnp.float32),
        pltpu.SemaphoreType.DMA,
        pltpu.SemaphoreType.DMA,
        pltpu.SemaphoreType.DMA,
        pltpu.SemaphoreType.DMA,
        pltpu.SemaphoreType.DMA,
        pltpu.SemaphoreType.DMA,
    ],
)


@jax.jit
def kernel(x_list, policy_list, layer_token0, pos_embed):
  pol = jnp.pad(policy_list, ((0, 0), (0, LPAD - L))).reshape(B, NROW, NCOL)
  idx = _index(pol)
  return _gather(x_list, idx, layer_token0, pos_embed)
